# baseline (speedup 1.0000x reference)
import functools

import jax
import jax.numpy as jnp
from jax import lax
from jax.experimental import pallas as pl
from jax.experimental.pallas import tpu as pltpu
from jax.experimental.pallas import tpu_sc as plsc

_D = 5
_ETA_MIN = 0.8
_ETA_RANGE = 0.99 - 0.8

_B, _T = 16384, 200
_N = _B * _T
_V = 100000
_FLAT = _V * _D
_FPAD = 3907 * 128

_NC, _NS = 2, 16
_NW = _NC * _NS
_TT = _T // 8
_BB = _B // 128
_QT = 2
_CHIDS = 8 * 128 * _QT
_NCH = _TT * _BB // _QT
_CPW = _NCH // _NW


def _sigmoid_body(x_ref, o_ref):
    x = x_ref[...]
    o_ref[...] = _ETA_MIN + _ETA_RANGE / (1.0 + jnp.exp(-x))


_transform = pl.pallas_call(
    _sigmoid_body,
    out_shape=jax.ShapeDtypeStruct((_FPAD // 128, 128), jnp.float32),
)

_sc_mesh = plsc.VectorSubcoreMesh(core_axis_name="c", subcore_axis_name="s")


@functools.partial(
    pl.kernel,
    mesh=_sc_mesh,
    out_type=jax.ShapeDtypeStruct((_D * _N,), jnp.float32),
    scratch_types=[
        pltpu.VMEM((2, _CHIDS), jnp.int32),
        pltpu.VMEM((2, _D * _CHIDS), jnp.float32),
        pltpu.VMEM_SHARED((_D, _V), jnp.float32),
        pltpu.SemaphoreType.DMA,
        pltpu.SemaphoreType.DMA,
    ],
    compiler_params=pltpu.CompilerParams(use_tc_tiling_on_sc=False),
)
def _gather_kernel(table_hbm, idx_hbm, out_hbm, idx_v, tile_v, table_sp,
                   gsem, wsem):
    sid = lax.axis_index("s")
    wid = sid * _NC + lax.axis_index("c")

    @pl.when(sid == 0)
    def _stage():
        pltpu.sync_copy(table_hbm, table_sp)

    plsc.subcore_barrier()

    def do_gathers(ct, buf):
        pltpu.sync_copy(idx_hbm.at[pl.ds(ct * _CHIDS, _CHIDS)],
                        idx_v.at[buf])
        gathers = [
            pltpu.async_copy(table_sp.at[c].at[idx_v.at[buf]],
                             tile_v.at[buf].at[pl.ds(c * _CHIDS, _CHIDS)],
                             gsem)
            for c in range(_D)
        ]
        for g in gathers:
            g.wait()

    def do_writes(ct, buf):
        nq = _BB // _QT
        tt = ct // nq
        bb = (ct - tt * nq) * _QT
        return [
            pltpu.async_copy(
                tile_v.at[buf].at[pl.ds(c * _CHIDS, _CHIDS)],
                out_hbm.at[pl.ds(((c * _TT + tt) * _BB + bb) * 1024,
                                 _CHIDS)], wsem)
            for c in range(_D)
        ]

    def pair(j, carry):
        ct_a = wid * _CPW + 2 * j
        ct_b = ct_a + 1
        do_gathers(ct_a, 0)
        writes_a = do_writes(ct_a, 0)
        do_gathers(ct_b, 1)
        writes_b = do_writes(ct_b, 1)
        for w in writes_a:
            w.wait()
        for w in writes_b:
            w.wait()
        return carry

    lax.fori_loop(0, _CPW // 2, pair, 0)


def kernel(ops_t, cond_ids, eta_table):
    del ops_t
    flat_cm = jnp.pad(eta_table.T.reshape(-1), (0, _FPAD - _FLAT))
    table = _transform(flat_cm.reshape(-1, 128)).reshape(-1)[:_FLAT]
    table = table.reshape(_D, _V)
    idx = (cond_ids.T.reshape(_TT, 8, _BB, 128)
           .transpose(0, 2, 1, 3).reshape(_N))
    out = _gather_kernel(table, idx)
    out = out.reshape(_D, _TT, _BB, 8, 128).transpose(2, 4, 1, 3, 0)
    return out.reshape(_B, _T, _D)


# speedup vs baseline: 1.0754x; 1.0754x over previous
import functools

import jax
import jax.numpy as jnp
from jax import lax
from jax.experimental import pallas as pl
from jax.experimental.pallas import tpu as pltpu
from jax.experimental.pallas import tpu_sc as plsc

_D = 5
_ETA_MIN = 0.8
_ETA_RANGE = 0.99 - 0.8

_B, _T = 16384, 200
_N = _B * _T
_V = 100000
_FLAT = _V * _D
_FPAD = 3907 * 128

_NC, _NS = 2, 16
_NW = _NC * _NS
_TT = _T // 8
_BB = _B // 128
_QT = 4
_CHIDS = 8 * 128 * _QT
_NCH = _TT * _BB // _QT
_CPW = _NCH // _NW


def _sigmoid_body(x_ref, o_ref):
    x = x_ref[...]
    o_ref[...] = _ETA_MIN + _ETA_RANGE / (1.0 + jnp.exp(-x))


_transform = pl.pallas_call(
    _sigmoid_body,
    out_shape=jax.ShapeDtypeStruct((_FPAD // 128, 128), jnp.float32),
)

_sc_mesh = plsc.VectorSubcoreMesh(core_axis_name="c", subcore_axis_name="s")


@functools.partial(
    pl.kernel,
    mesh=_sc_mesh,
    out_type=jax.ShapeDtypeStruct((_D * _N,), jnp.float32),
    scratch_types=[
        pltpu.VMEM((2, _CHIDS), jnp.int32),
        pltpu.VMEM((2, _D * _CHIDS), jnp.float32),
        pltpu.VMEM_SHARED((_D, _V), jnp.float32),
        pltpu.SemaphoreType.DMA,
        pltpu.SemaphoreType.DMA,
    ],
    compiler_params=pltpu.CompilerParams(use_tc_tiling_on_sc=False),
)
def _gather_kernel(table_hbm, idx_hbm, out_hbm, idx_v, tile_v, table_sp,
                   gsem, wsem):
    sid = lax.axis_index("s")
    wid = sid * _NC + lax.axis_index("c")

    @pl.when(sid == 0)
    def _stage():
        pltpu.sync_copy(table_hbm, table_sp)

    plsc.subcore_barrier()

    def do_gathers(ct, buf):
        pltpu.sync_copy(idx_hbm.at[pl.ds(ct * _CHIDS, _CHIDS)],
                        idx_v.at[buf])
        gathers = [
            pltpu.async_copy(table_sp.at[c].at[idx_v.at[buf]],
                             tile_v.at[buf].at[pl.ds(c * _CHIDS, _CHIDS)],
                             gsem)
            for c in range(_D)
        ]
        for g in gathers:
            g.wait()

    def do_writes(ct, buf):
        nq = _BB // _QT
        tt = ct // nq
        bb = (ct - tt * nq) * _QT
        return [
            pltpu.async_copy(
                tile_v.at[buf].at[pl.ds(c * _CHIDS, _CHIDS)],
                out_hbm.at[pl.ds(((c * _TT + tt) * _BB + bb) * 1024,
                                 _CHIDS)], wsem)
            for c in range(_D)
        ]

    def pair(j, carry):
        ct_a = wid * _CPW + 2 * j
        ct_b = ct_a + 1
        do_gathers(ct_a, 0)
        writes_a = do_writes(ct_a, 0)
        do_gathers(ct_b, 1)
        writes_b = do_writes(ct_b, 1)
        for w in writes_a:
            w.wait()
        for w in writes_b:
            w.wait()
        return carry

    lax.fori_loop(0, _CPW // 2, pair, 0)
    if _CPW % 2:
        ct_last = wid * _CPW + (_CPW - 1)
        do_gathers(ct_last, 0)
        for w in do_writes(ct_last, 0):
            w.wait()


def kernel(ops_t, cond_ids, eta_table):
    del ops_t
    flat_cm = jnp.pad(eta_table.T.reshape(-1), (0, _FPAD - _FLAT))
    table = _transform(flat_cm.reshape(-1, 128)).reshape(-1)[:_FLAT]
    table = table.reshape(_D, _V)
    idx = (cond_ids.T.reshape(_TT, 8, _BB, 128)
           .transpose(0, 2, 1, 3).reshape(_N))
    out = _gather_kernel(table, idx)
    out = out.reshape(_D, _TT, _BB, 8, 128).transpose(2, 4, 1, 3, 0)
    return out.reshape(_B, _T, _D)
